# baseline (speedup 1.0000x reference)
import functools

import jax
import jax.numpy as jnp
from jax import lax
from jax.experimental import pallas as pl
from jax.experimental.pallas import tpu as pltpu, tpu_sc as plsc

N_NODES = 100000
N_EDGES = 6400000
ROW = 8

NUM_CORES = 2
NUM_SUBCORES = 16
LANES = 16
NW = NUM_CORES * NUM_SUBCORES

E_PER_W = N_EDGES // NW
CHUNK = 800
N_CHUNKS = E_PER_W // CHUNK
NPHASE = 3
ROUNDS = (N_CHUNKS + NPHASE - 1) // NPHASE

_VEC_SCRATCH = [
    pltpu.VMEM((CHUNK,), jnp.int32),
    pltpu.VMEM((CHUNK,), jnp.int32),
    pltpu.VMEM((CHUNK, ROW), jnp.float32),
    pltpu.VMEM((CHUNK, ROW), jnp.float32),
    pltpu.VMEM((CHUNK,), jnp.float32),
    pltpu.VMEM((CHUNK,), jnp.float32),
    pltpu.VMEM((CHUNK,), jnp.float32),
    pltpu.VMEM((CHUNK,), jnp.float32),
    pltpu.VMEM((CHUNK,), jnp.float32),
    pltpu.VMEM((CHUNK,), jnp.float32),
]


def _edge_kernel(pos_hbm, ii_hbm, ij_hbm, o0_hbm, o1_hbm, o2_hbm,
                 u0_hbm, u1_hbm, u2_hbm, tab_s, *scratch):
    sid = lax.axis_index("s")
    wid = sid * NUM_CORES + lax.axis_index("c")
    w_base = wid * E_PER_W

    nper = len(_VEC_SCRATCH)
    sem = scratch[-1]
    sets = [scratch[p * nper:(p + 1) * nper] + (sem.at[p, 0], sem.at[p, 1],
                                                sem.at[p, 2], sem.at[p, 3])
            for p in range(NPHASE)]

    @pl.when(sid == 0)
    def _():
        pltpu.sync_copy(pos_hbm, tab_s)

    plsc.subcore_barrier()

    def clamp(c):
        return jnp.minimum(c, N_CHUNKS - 1)

    def in_copies(s, c):
        (ii_v, ij_v, _, _, b0, b1, b2, _, _, _,
         sem_idx, sem_off, _, _) = s
        ld = pl.ds(w_base + clamp(c) * CHUNK, CHUNK)
        return (
            (pltpu.make_async_copy(ii_hbm.at[ld], ii_v, sem_idx),
             pltpu.make_async_copy(ij_hbm.at[ld], ij_v, sem_idx)),
            (pltpu.make_async_copy(o0_hbm.at[ld], b0, sem_off),
             pltpu.make_async_copy(o1_hbm.at[ld], b1, sem_off),
             pltpu.make_async_copy(o2_hbm.at[ld], b2, sem_off)),
        )

    def out_copies(s, c):
        (_, _, _, _, _, _, _, r0, r1, r2, _, _, _, sem_out) = s
        ld = pl.ds(w_base + clamp(c) * CHUNK, CHUNK)
        return (pltpu.make_async_copy(r0, u0_hbm.at[ld], sem_out),
                pltpu.make_async_copy(r1, u1_hbm.at[ld], sem_out),
                pltpu.make_async_copy(r2, u2_hbm.at[ld], sem_out))

    def gather_copies(s):
        (ii_v, ij_v, gi_v, gj_v, _, _, _, _, _, _, _, _, sem_g, _) = s
        return (pltpu.make_async_copy(tab_s.at[ii_v], gi_v, sem_g),
                pltpu.make_async_copy(tab_s.at[ij_v], gj_v, sem_g))

    def start_in(s, c):
        idx, off = in_copies(s, c)
        for cp in idx + off:
            cp.start()

    def wait_idx(s, c):
        for cp in in_copies(s, c)[0]:
            cp.wait()

    def wait_off(s, c):
        for cp in in_copies(s, c)[1]:
            cp.wait()

    def start_gather(s):
        for cp in gather_copies(s):
            cp.start()

    def wait_gather(s):
        for cp in gather_copies(s):
            cp.wait()

    def compute(s):
        (_, _, gi_v, gj_v, b0, b1, b2, r0, r1, r2, _, _, _, _) = s
        bs = (b0, b1, b2)
        rs = (r0, r1, r2)

        def vec_body(t, carry):
            e0 = t * LANES
            ev = e0 + lax.iota(jnp.int32, LANES)
            for k in range(3):
                ck = jnp.full((LANES,), k, jnp.int32)
                pi = plsc.load_gather(gi_v, [ev, ck])
                pj = plsc.load_gather(gj_v, [ev, ck])
                rs[k][pl.ds(e0, LANES)] = bs[k][pl.ds(e0, LANES)] + pj - pi
            return carry

        lax.fori_loop(0, CHUNK // LANES, vec_body, 0, unroll=4)

    def phase(p, s, g3, c):
        wait_gather(s)
        wait_off(s, c)

        @pl.when(g3 > 0)
        def _():
            for cp in out_copies(s, c - NPHASE):
                cp.wait()

        compute(s)
        for cp in out_copies(s, c):
            cp.start()
        start_in(s, c + NPHASE)

    start_in(sets[0], 0)
    start_in(sets[1], 1)
    start_in(sets[2], 2)
    wait_idx(sets[0], 0)
    start_gather(sets[0])

    def round_body(g3, carry):
        cA = g3 * NPHASE
        wait_idx(sets[1], cA + 1)
        start_gather(sets[1])
        phase(0, sets[0], g3, cA)
        wait_idx(sets[2], cA + 2)
        start_gather(sets[2])
        phase(1, sets[1], g3, cA + 1)
        wait_idx(sets[0], cA + 3)
        start_gather(sets[0])
        phase(2, sets[2], g3, cA + 2)
        return carry

    lax.fori_loop(0, ROUNDS, round_body, 0)

    last = ROUNDS * NPHASE
    wait_gather(sets[0])
    wait_off(sets[0], last)
    wait_idx(sets[1], last + 1)
    wait_off(sets[1], last + 1)
    wait_idx(sets[2], last + 2)
    wait_off(sets[2], last + 2)
    for p in range(NPHASE):
        for cp in out_copies(sets[p], last - NPHASE + p):
            cp.wait()


@jax.jit
def kernel(positions, indeces_i, indeces_j, offsets):
    mesh = plsc.VectorSubcoreMesh(core_axis_name="c", subcore_axis_name="s")
    vec = jax.ShapeDtypeStruct((N_EDGES,), jnp.float32)
    run = pl.kernel(
        _edge_kernel,
        out_type=(vec, vec, vec),
        mesh=mesh,
        compiler_params=pltpu.CompilerParams(
            needs_layout_passes=False, use_tc_tiling_on_sc=False),
        scratch_types=(
            [pltpu.VMEM_SHARED((N_NODES, ROW), jnp.float32)]
            + _VEC_SCRATCH * NPHASE
            + [pltpu.SemaphoreType.DMA((NPHASE, 4))]
        ),
    )
    pos_pad = jnp.pad(positions, ((0, 0), (0, ROW - 3)))
    u0, u1, u2 = run(
        pos_pad,
        indeces_i.astype(jnp.int32),
        indeces_j.astype(jnp.int32),
        offsets[:, 0],
        offsets[:, 1],
        offsets[:, 2],
    )
    return jnp.stack([u0, u1, u2], axis=1)
